# baseline (speedup 1.0000x reference)
import functools

import jax
import jax.numpy as jnp
from jax import lax
from jax.experimental import pallas as pl
from jax.experimental.pallas import tpu as pltpu
from jax.experimental.pallas import tpu_sc as plsc

_NC = 2
_NS = 16
_NW = _NC * _NS
_L = 16


@functools.partial(jax.jit, static_argnums=(4, 5))
def _lookup(ids, seg, emb, comb, T, H):
    N = ids.shape[0]
    C = 32
    n_chunk = T // C
    n_vec = H // _L

    mesh = plsc.VectorSubcoreMesh(core_axis_name="c", subcore_axis_name="s")

    @functools.partial(
        pl.kernel,
        mesh=mesh,
        out_type=jax.ShapeDtypeStruct((N, H), jnp.float32),
        scratch_types=[
            pltpu.VMEM((T,), jnp.int32),
            pltpu.VMEM((T,), jnp.int32),
            pltpu.VMEM((C, H), jnp.float32),
            pltpu.VMEM((C, H), jnp.float32),
            pltpu.SemaphoreType.DMA,
            pltpu.SemaphoreType.DMA,
        ],
    )
    def body(ids_hbm, seg_hbm, emb_hbm, comb_hbm, out_hbm,
             ids_v, idx2_v, bufA, bufB, semA, semB):
        wid = lax.axis_index("s") * _NC + lax.axis_index("c")
        base = wid * T
        pltpu.sync_copy(ids_hbm.at[pl.ds(base, T)], ids_v)
        pltpu.sync_copy(seg_hbm.at[pl.ds(base, T)], idx2_v)

        iota2 = lax.iota(jnp.int32, _L) * 2

        def idx_body(k, carry):
            sv = idx2_v[pl.ds(k * _L, _L)]
            idx2_v[pl.ds(k * _L, _L)] = sv + iota2 + (2 * _L) * k
            return carry

        lax.fori_loop(0, T // _L, idx_body, 0)

        def chunk_body(c, carry):
            row0 = c * C
            ga = pltpu.async_copy(emb_hbm.at[ids_v.at[pl.ds(row0, C)]], bufA, semA)
            gb = pltpu.async_copy(comb_hbm.at[idx2_v.at[pl.ds(row0, C)]], bufB, semB)
            ga.wait()
            gb.wait()

            def add_row(i, carry2):
                def add_vec(j, carry3):
                    x = bufB[i, pl.ds(j * _L, _L)]
                    plsc.addupdate(bufA.at[i, pl.ds(j * _L, _L)], x)
                    return carry3
                return lax.fori_loop(0, n_vec, add_vec, carry2)

            lax.fori_loop(0, C, add_row, 0)
            pltpu.sync_copy(bufA, out_hbm.at[pl.ds(base + row0, C)])
            return carry

        lax.fori_loop(0, n_chunk, chunk_body, 0)

    return body(ids, seg, emb, comb)


def kernel(input_ids, segment_ids, embedding_table, token_type_table,
           full_position_embeddings):
    B, S = input_ids.shape
    H = embedding_table.shape[1]
    comb = (full_position_embeddings[:S, None, :]
            + token_type_table[None, :, :]).reshape(2 * S, H)
    ids = input_ids.reshape(-1).astype(jnp.int32)
    seg = segment_ids.reshape(-1).astype(jnp.int32)
    T = (B * S) // _NW
    out = _lookup(ids, seg, embedding_table, comb, T, H)
    return out.reshape(B, S, H)


# speedup vs baseline: 1.1732x; 1.1732x over previous
import functools

import jax
import jax.numpy as jnp
from jax import lax
from jax.experimental import pallas as pl
from jax.experimental.pallas import tpu as pltpu
from jax.experimental.pallas import tpu_sc as plsc

_NC = 2
_NS = 16
_NW = _NC * _NS
_L = 16


@functools.partial(jax.jit, static_argnums=(4, 5))
def _lookup(ids, seg, emb, comb, T, H):
    N = ids.shape[0]
    C = 32
    n_chunk = T // C
    n_vec = H // _L

    mesh = plsc.VectorSubcoreMesh(core_axis_name="c", subcore_axis_name="s")

    @functools.partial(
        pl.kernel,
        mesh=mesh,
        out_type=jax.ShapeDtypeStruct((N, H), jnp.float32),
        scratch_types=[
            pltpu.VMEM((T,), jnp.int32),
            pltpu.VMEM((T,), jnp.int32),
            pltpu.VMEM((C, H), jnp.float32),
            pltpu.VMEM((C, H), jnp.float32),
            pltpu.SemaphoreType.DMA,
            pltpu.SemaphoreType.DMA,
        ],
    )
    def body(ids_hbm, seg_hbm, emb_hbm, comb_hbm, out_hbm,
             ids_v, idx2_v, bufA, bufB, semA, semB):
        wid = lax.axis_index("s") * _NC + lax.axis_index("c")
        base = wid * T
        pltpu.sync_copy(ids_hbm.at[pl.ds(base, T)], ids_v)
        pltpu.sync_copy(seg_hbm.at[pl.ds(base, T)], idx2_v)

        iota2 = lax.iota(jnp.int32, _L) * 2

        def idx_body(k, carry):
            sv = idx2_v[pl.ds(k * _L, _L)]
            idx2_v[pl.ds(k * _L, _L)] = sv + iota2 + (2 * _L) * k
            return carry

        lax.fori_loop(0, T // _L, idx_body, 0)

        def chunk_body(c, carry):
            row0 = c * C
            ga = pltpu.async_copy(emb_hbm.at[ids_v.at[pl.ds(row0, C)]], bufA, semA)
            gb = pltpu.async_copy(comb_hbm.at[idx2_v.at[pl.ds(row0, C)]], bufB, semB)
            ga.wait()
            gb.wait()

            def add_row(i, carry2):
                for j in range(n_vec):
                    x = bufB[i, pl.ds(j * _L, _L)]
                    plsc.addupdate(bufA.at[i, pl.ds(j * _L, _L)], x)
                return carry2

            lax.fori_loop(0, C, add_row, 0)
            pltpu.sync_copy(bufA, out_hbm.at[pl.ds(base + row0, C)])
            return carry

        lax.fori_loop(0, n_chunk, chunk_body, 0)

    return body(ids, seg, emb, comb)


def kernel(input_ids, segment_ids, embedding_table, token_type_table,
           full_position_embeddings):
    B, S = input_ids.shape
    H = embedding_table.shape[1]
    comb = (full_position_embeddings[:S, None, :]
            + token_type_table[None, :, :]).reshape(2 * S, H)
    ids = input_ids.reshape(-1).astype(jnp.int32)
    seg = segment_ids.reshape(-1).astype(jnp.int32)
    T = (B * S) // _NW
    out = _lookup(ids, seg, embedding_table, comb, T, H)
    return out.reshape(B, S, H)


# speedup vs baseline: 1.7371x; 1.4806x over previous
import functools

import jax
import jax.numpy as jnp
from jax import lax
from jax.experimental import pallas as pl
from jax.experimental.pallas import tpu as pltpu
from jax.experimental.pallas import tpu_sc as plsc

_NC = 2
_NS = 16
_NW = _NC * _NS
_L = 16


@functools.partial(jax.jit, static_argnums=(4, 5))
def _lookup(ids, seg, emb, comb, T, H):
    N = ids.shape[0]
    C = 32
    n_chunk = T // C
    n_vec = H // _L

    mesh = plsc.VectorSubcoreMesh(core_axis_name="c", subcore_axis_name="s")

    @functools.partial(
        pl.kernel,
        mesh=mesh,
        out_type=jax.ShapeDtypeStruct((N, H), jnp.float32),
        scratch_types=[
            pltpu.VMEM((T,), jnp.int32),
            pltpu.VMEM((T,), jnp.int32),
            pltpu.VMEM((C, H), jnp.float32),
            pltpu.VMEM((C, H), jnp.float32),
            pltpu.SemaphoreType.DMA,
            pltpu.SemaphoreType.DMA,
        ],
    )
    def body(ids_hbm, seg_hbm, emb_hbm, comb_hbm, out_hbm,
             ids_v, idx2_v, bufA, bufB, semA, semB):
        wid = lax.axis_index("s") * _NC + lax.axis_index("c")
        base = wid * T
        pltpu.sync_copy(ids_hbm.at[pl.ds(base, T)], ids_v)
        pltpu.sync_copy(seg_hbm.at[pl.ds(base, T)], idx2_v)

        iota2 = lax.iota(jnp.int32, _L) * 2

        def idx_body(k, carry):
            sv = idx2_v[pl.ds(k * _L, _L)]
            idx2_v[pl.ds(k * _L, _L)] = sv + iota2 + (2 * _L) * k
            return carry

        lax.fori_loop(0, T // _L, idx_body, 0)

        def chunk_body(c, carry):
            row0 = c * C
            ga = pltpu.async_copy(emb_hbm.at[ids_v.at[pl.ds(row0, C)]], bufA, semA)
            gb = pltpu.async_copy(comb_hbm.at[idx2_v.at[pl.ds(row0, C)]], bufB, semB)
            ga.wait()
            gb.wait()

            @plsc.parallel_loop(0, C * n_vec, unroll=8)
            def add_vec(k):
                i = k // n_vec
                j = k % n_vec
                x = bufB[i, pl.ds(j * _L, _L)]
                plsc.addupdate(bufA.at[i, pl.ds(j * _L, _L)], x)
            pltpu.sync_copy(bufA, out_hbm.at[pl.ds(base + row0, C)])
            return carry

        lax.fori_loop(0, n_chunk, chunk_body, 0)

    return body(ids, seg, emb, comb)


def kernel(input_ids, segment_ids, embedding_table, token_type_table,
           full_position_embeddings):
    B, S = input_ids.shape
    H = embedding_table.shape[1]
    comb = (full_position_embeddings[:S, None, :]
            + token_type_table[None, :, :]).reshape(2 * S, H)
    ids = input_ids.reshape(-1).astype(jnp.int32)
    seg = segment_ids.reshape(-1).astype(jnp.int32)
    T = (B * S) // _NW
    out = _lookup(ids, seg, embedding_table, comb, T, H)
    return out.reshape(B, S, H)


# speedup vs baseline: 2.2539x; 1.2975x over previous
import functools

import jax
import jax.numpy as jnp
from jax import lax
from jax.experimental import pallas as pl
from jax.experimental.pallas import tpu as pltpu
from jax.experimental.pallas import tpu_sc as plsc

_NC = 2
_NS = 16
_NW = _NC * _NS
_L = 16


@functools.partial(jax.jit, static_argnums=(4, 5))
def _lookup(ids, seg, emb, comb, T, H):
    N = ids.shape[0]
    C = 16
    n_chunk = T // C
    n_pair = n_chunk // 2
    n_vec = H // _L

    mesh = plsc.VectorSubcoreMesh(core_axis_name="c", subcore_axis_name="s")

    @functools.partial(
        pl.kernel,
        mesh=mesh,
        out_type=jax.ShapeDtypeStruct((N, H), jnp.float32),
        scratch_types=[
            pltpu.VMEM((T,), jnp.int32),
            pltpu.VMEM((T,), jnp.int32),
            pltpu.VMEM((C, H), jnp.float32),
            pltpu.VMEM((C, H), jnp.float32),
            pltpu.VMEM((C, H), jnp.float32),
            pltpu.VMEM((C, H), jnp.float32),
            pltpu.VMEM((C, H), jnp.float32),
            pltpu.VMEM((C, H), jnp.float32),
            pltpu.SemaphoreType.DMA,
            pltpu.SemaphoreType.DMA,
            pltpu.SemaphoreType.DMA,
            pltpu.SemaphoreType.DMA,
            pltpu.SemaphoreType.DMA,
            pltpu.SemaphoreType.DMA,
        ],
    )
    def body(ids_hbm, seg_hbm, emb_hbm, comb_hbm, out_hbm,
             ids_v, idx2_v, bufA0, bufB0, bufC0, bufA1, bufB1, bufC1,
             semA0, semB0, semO0, semA1, semB1, semO1):
        slots = ((bufA0, bufB0, bufC0, semA0, semB0, semO0),
                 (bufA1, bufB1, bufC1, semA1, semB1, semO1))
        wid = lax.axis_index("s") * _NC + lax.axis_index("c")
        base = wid * T
        pltpu.sync_copy(ids_hbm.at[pl.ds(base, T)], ids_v)
        pltpu.sync_copy(seg_hbm.at[pl.ds(base, T)], idx2_v)

        iota2 = lax.iota(jnp.int32, _L) * 2

        def idx_body(k, carry):
            sv = idx2_v[pl.ds(k * _L, _L)]
            idx2_v[pl.ds(k * _L, _L)] = sv + iota2 + (2 * _L) * k
            return carry

        lax.fori_loop(0, T // _L, idx_body, 0)

        def issue_gathers(c, slot):
            bufA, bufB, _, semA, semB, _ = slot
            row0 = c * C
            pltpu.async_copy(emb_hbm.at[ids_v.at[pl.ds(row0, C)]], bufA, semA)
            pltpu.async_copy(comb_hbm.at[idx2_v.at[pl.ds(row0, C)]], bufB, semB)

        issue_gathers(0, slots[0])
        issue_gathers(1, slots[1])

        def pair_body(p, carry):
            for s in range(2):
                bufA, bufB, bufC, semA, semB, semO = slots[s]
                c = 2 * p + s
                row0 = c * C
                pltpu.make_async_copy(emb_hbm.at[pl.ds(0, C)], bufA, semA).wait()
                pltpu.make_async_copy(comb_hbm.at[pl.ds(0, C)], bufB, semB).wait()

                @pl.when(p >= 1)
                def _():
                    pltpu.make_async_copy(
                        bufC, out_hbm.at[pl.ds(base, C)], semO).wait()

                @plsc.parallel_loop(0, C * n_vec, unroll=8)
                def add_vec(k):
                    i = k // n_vec
                    j = k % n_vec
                    bufC[i, pl.ds(j * _L, _L)] = (
                        bufA[i, pl.ds(j * _L, _L)] + bufB[i, pl.ds(j * _L, _L)])

                pltpu.async_copy(bufC, out_hbm.at[pl.ds(base + row0, C)], semO)

                @pl.when(p < n_pair - 1)
                def _():
                    issue_gathers(c + 2, slots[s])
            return carry

        lax.fori_loop(0, n_pair, pair_body, 0)
        pltpu.make_async_copy(bufC0, out_hbm.at[pl.ds(base, C)], semO0).wait()
        pltpu.make_async_copy(bufC1, out_hbm.at[pl.ds(base, C)], semO1).wait()

    return body(ids, seg, emb, comb)


def kernel(input_ids, segment_ids, embedding_table, token_type_table,
           full_position_embeddings):
    B, S = input_ids.shape
    H = embedding_table.shape[1]
    comb = (full_position_embeddings[:S, None, :]
            + token_type_table[None, :, :]).reshape(2 * S, H)
    ids = input_ids.reshape(-1).astype(jnp.int32)
    seg = segment_ids.reshape(-1).astype(jnp.int32)
    T = (B * S) // _NW
    out = _lookup(ids, seg, embedding_table, comb, T, H)
    return out.reshape(B, S, H)


# speedup vs baseline: 2.2562x; 1.0010x over previous
import functools

import jax
import jax.numpy as jnp
from jax import lax
from jax.experimental import pallas as pl
from jax.experimental.pallas import tpu as pltpu
from jax.experimental.pallas import tpu_sc as plsc

_NC = 2
_NS = 16
_NW = _NC * _NS
_L = 16


@functools.partial(jax.jit, static_argnums=(4, 5))
def _lookup(ids, seg, emb, comb, T, H):
    N = ids.shape[0]
    C = 16
    n_chunk = T // C
    n_quad = n_chunk // 4
    n_vec = H // _L

    mesh = plsc.VectorSubcoreMesh(core_axis_name="c", subcore_axis_name="s")

    @functools.partial(
        pl.kernel,
        mesh=mesh,
        out_type=jax.ShapeDtypeStruct((N, H), jnp.float32),
        scratch_types=[
            pltpu.VMEM((T,), jnp.int32),
            pltpu.VMEM((T,), jnp.int32),
            pltpu.VMEM((C, H), jnp.float32),
            pltpu.VMEM((C, H), jnp.float32),
            pltpu.VMEM((C, H), jnp.float32),
            pltpu.VMEM((C, H), jnp.float32),
            pltpu.VMEM((C, H), jnp.float32),
            pltpu.VMEM((C, H), jnp.float32),
            pltpu.SemaphoreType.DMA,
            pltpu.SemaphoreType.DMA,
            pltpu.SemaphoreType.DMA,
            pltpu.SemaphoreType.DMA,
            pltpu.SemaphoreType.DMA,
            pltpu.SemaphoreType.DMA,
            pltpu.SemaphoreType.DMA,
            pltpu.SemaphoreType.DMA,
            pltpu.SemaphoreType.DMA,
            pltpu.SemaphoreType.DMA,
        ],
    )
    def body(ids_hbm, seg_hbm, emb_hbm, comb_hbm, out_hbm,
             ids_v, idx2_v, embA0, embA1, accX0, accX1, accX2, accX3,
             semA0, semA1, semX0, semX1, semX2, semX3,
             semO0, semO1, semO2, semO3):
        embs = ((embA0, semA0), (embA1, semA1))
        accs = ((accX0, semX0, semO0), (accX1, semX1, semO1),
                (accX2, semX2, semO2), (accX3, semX3, semO3))
        wid = lax.axis_index("s") * _NC + lax.axis_index("c")
        base = wid * T
        pltpu.sync_copy(ids_hbm.at[pl.ds(base, T)], ids_v)
        pltpu.sync_copy(seg_hbm.at[pl.ds(base, T)], idx2_v)

        iota2 = lax.iota(jnp.int32, _L) * 2

        def idx_body(k, carry):
            sv = idx2_v[pl.ds(k * _L, _L)]
            idx2_v[pl.ds(k * _L, _L)] = sv + iota2 + (2 * _L) * k
            return carry

        lax.fori_loop(0, T // _L, idx_body, 0)

        def issue_emb(c, u):
            bufA, semA = embs[u % 2]
            pltpu.async_copy(emb_hbm.at[ids_v.at[pl.ds(c * C, C)]], bufA, semA)

        def issue_comb(c, u):
            bufX, semX, _ = accs[u % 4]
            pltpu.async_copy(comb_hbm.at[idx2_v.at[pl.ds(c * C, C)]], bufX, semX)

        issue_emb(0, 0)
        issue_comb(0, 0)
        issue_emb(1, 1)
        issue_comb(1, 1)

        def quad_body(q, carry):
            for u in range(4):
                bufA, semA = embs[u % 2]
                bufX, semX, semO = accs[u]
                c = 4 * q + u
                pltpu.make_async_copy(emb_hbm.at[pl.ds(0, C)], bufA, semA).wait()
                pltpu.make_async_copy(comb_hbm.at[pl.ds(0, C)], bufX, semX).wait()

                @plsc.parallel_loop(0, C * n_vec, unroll=8)
                def add_vec(k):
                    i = k // n_vec
                    j = k % n_vec
                    x = bufA[i, pl.ds(j * _L, _L)]
                    plsc.addupdate(bufX.at[i, pl.ds(j * _L, _L)], x)

                pltpu.async_copy(bufX, out_hbm.at[pl.ds(base + c * C, C)], semO)

                @pl.when(c + 2 < n_chunk)
                def _():
                    issue_emb(c + 2, u + 2)
                    bufX2, _, semO2_ = accs[(u + 2) % 4]

                    @pl.when(c - 2 >= 0)
                    def _():
                        pltpu.make_async_copy(
                            bufX2, out_hbm.at[pl.ds(base, C)], semO2_).wait()

                    issue_comb(c + 2, u + 2)
            return carry

        lax.fori_loop(0, n_quad, quad_body, 0)
        pltpu.make_async_copy(accX0, out_hbm.at[pl.ds(base, C)], semO0).wait()
        pltpu.make_async_copy(accX1, out_hbm.at[pl.ds(base, C)], semO1).wait()
        pltpu.make_async_copy(accX2, out_hbm.at[pl.ds(base, C)], semO2).wait()
        pltpu.make_async_copy(accX3, out_hbm.at[pl.ds(base, C)], semO3).wait()

    return body(ids, seg, emb, comb)


def kernel(input_ids, segment_ids, embedding_table, token_type_table,
           full_position_embeddings):
    B, S = input_ids.shape
    H = embedding_table.shape[1]
    comb = (full_position_embeddings[:S, None, :]
            + token_type_table[None, :, :]).reshape(2 * S, H)
    ids = input_ids.reshape(-1).astype(jnp.int32)
    seg = segment_ids.reshape(-1).astype(jnp.int32)
    T = (B * S) // _NW
    out = _lookup(ids, seg, embedding_table, comb, T, H)
    return out.reshape(B, S, H)


# speedup vs baseline: 2.9150x; 1.2920x over previous
import functools

import jax
import jax.numpy as jnp
from jax import lax
from jax.experimental import pallas as pl
from jax.experimental.pallas import tpu as pltpu
from jax.experimental.pallas import tpu_sc as plsc

_NC = 2
_NS = 16
_NW = _NC * _NS
_L = 16


@functools.partial(jax.jit, static_argnums=(5, 6))
def _lookup(ids, seg, emb, pos, tt, B, S):
    H = emb.shape[1]
    G = S // _NW
    C = G
    n_chunk = B
    n_quad = n_chunk // 4
    n_vec = H // _L

    mesh = plsc.VectorSubcoreMesh(core_axis_name="c", subcore_axis_name="s")

    @functools.partial(
        pl.kernel,
        mesh=mesh,
        out_type=jax.ShapeDtypeStruct((B * S, H), jnp.float32),
        scratch_types=[
            pltpu.VMEM((B * C,), jnp.int32),
            pltpu.VMEM((B * C,), jnp.int32),
            pltpu.VMEM((2, H), jnp.float32),
            pltpu.VMEM((2 * G, H), jnp.float32),
            pltpu.VMEM((C, H), jnp.float32),
            pltpu.VMEM((C, H), jnp.float32),
            pltpu.VMEM((C, H), jnp.float32),
            pltpu.VMEM((C, H), jnp.float32),
            pltpu.SemaphoreType.DMA,
            pltpu.SemaphoreType.DMA,
            pltpu.SemaphoreType.DMA,
            pltpu.SemaphoreType.DMA,
            pltpu.SemaphoreType.DMA,
            pltpu.SemaphoreType.DMA,
            pltpu.SemaphoreType.DMA,
            pltpu.SemaphoreType.DMA,
            pltpu.SemaphoreType.DMA,
        ],
    )
    def body(ids_hbm, seg_hbm, emb_hbm, pos_hbm, tt_hbm, out_hbm,
             idsb_v, segb_v, tt_v, ptt_v,
             accX0, accX1, accX2, accX3,
             semS, semX0, semX1, semX2, semX3, semO0, semO1, semO2, semO3):
        accs = ((accX0, semX0, semO0), (accX1, semX1, semO1),
                (accX2, semX2, semO2), (accX3, semX3, semO3))
        wid = lax.axis_index("s") * _NC + lax.axis_index("c")
        p0 = wid * G

        def pf_body(b, carry):
            src = b * S + p0
            pltpu.async_copy(ids_hbm.at[pl.ds(src, C)],
                             idsb_v.at[pl.ds(b * C, C)], semS)
            pltpu.async_copy(seg_hbm.at[pl.ds(src, C)],
                             segb_v.at[pl.ds(b * C, C)], semS)
            return carry

        lax.fori_loop(0, B, pf_body, 0)

        pltpu.sync_copy(pos_hbm.at[pl.ds(p0, G)], accX0)
        pltpu.sync_copy(tt_hbm, tt_v)

        def ptt_row(r, carry):
            @plsc.parallel_loop(0, n_vec, unroll=8)
            def ptt_vec(j):
                x = accX0[r // 2, pl.ds(j * _L, _L)] + tt_v[r % 2, pl.ds(j * _L, _L)]
                ptt_v[r, pl.ds(j * _L, _L)] = x
            return carry

        lax.fori_loop(0, 2 * G, ptt_row, 0)

        def pf_drain(b, carry):
            pltpu.make_async_copy(ids_hbm.at[pl.ds(0, C)],
                                  idsb_v.at[pl.ds(0, C)], semS).wait()
            pltpu.make_async_copy(seg_hbm.at[pl.ds(0, C)],
                                  segb_v.at[pl.ds(0, C)], semS).wait()
            return carry

        lax.fori_loop(0, B, pf_drain, 0)

        def issue_emb(c, u):
            bufX, semX, _ = accs[u % 4]
            pltpu.async_copy(emb_hbm.at[idsb_v.at[pl.ds(c * C, C)]], bufX, semX)

        issue_emb(0, 0)
        issue_emb(1, 1)

        def quad_body(q, carry):
            for u in range(4):
                bufX, semX, semO = accs[u]
                c = 4 * q + u
                pltpu.make_async_copy(emb_hbm.at[pl.ds(0, C)], bufX, semX).wait()

                rv = segb_v[pl.ds(c * C, _L)]
                for i in range(C):
                    r = 2 * i + rv[i]

                    @plsc.parallel_loop(0, n_vec, unroll=8)
                    def add_vec(j):
                        plsc.addupdate(bufX.at[i, pl.ds(j * _L, _L)],
                                       ptt_v[r, pl.ds(j * _L, _L)])

                pltpu.async_copy(bufX, out_hbm.at[pl.ds(c * S + p0, C)], semO)

                @pl.when(c + 2 < n_chunk)
                def _():
                    bufX2, _, semO2_ = accs[(u + 2) % 4]

                    @pl.when(c - 2 >= 0)
                    def _():
                        pltpu.make_async_copy(
                            bufX2, out_hbm.at[pl.ds(0, C)], semO2_).wait()

                    issue_emb(c + 2, u + 2)
            return carry

        lax.fori_loop(0, n_quad, quad_body, 0)
        pltpu.make_async_copy(accX0, out_hbm.at[pl.ds(0, C)], semO0).wait()
        pltpu.make_async_copy(accX1, out_hbm.at[pl.ds(0, C)], semO1).wait()
        pltpu.make_async_copy(accX2, out_hbm.at[pl.ds(0, C)], semO2).wait()
        pltpu.make_async_copy(accX3, out_hbm.at[pl.ds(0, C)], semO3).wait()

    return body(ids, seg, emb, pos, tt)


def kernel(input_ids, segment_ids, embedding_table, token_type_table,
           full_position_embeddings):
    B, S = input_ids.shape
    H = embedding_table.shape[1]
    ids = input_ids.reshape(-1).astype(jnp.int32)
    seg = segment_ids.reshape(-1).astype(jnp.int32)
    pos = full_position_embeddings[:S]
    out = _lookup(ids, seg, embedding_table, pos, token_type_table, B, S)
    return out.reshape(B, S, H)
